# baseline (speedup 1.0000x reference)
import functools

import jax
import jax.numpy as jnp
from jax import lax
from jax.experimental import pallas as pl
from jax.experimental.pallas import tpu as pltpu, tpu_sc as plsc

NC = 2
NS = 16
L = 16
B = 16384
D = 64
CHUNK = 128
DOT_ROWS = B // NS
DOT_CHUNKS = DOT_ROWS // CHUNK
OUT_ROWS = B // (NC * NS)
OUT_CHUNKS = OUT_ROWS // CHUNK


def _sc_body(uidx_hbm, midx_hbm, uemb_hbm, memb_hbm, ubias_hbm, mbias_hbm,
             gb_hbm, out_hbm,
             idx_u_v, idx_m_v, urows, mrows, ub_v, mb_v, out_v,
             gb_v, stage_v, part_v, shared, sem_u, sem_m):
    c = lax.axis_index("c")
    s = lax.axis_index("s")
    ot = c * NS + s

    acc = jnp.zeros((L,), jnp.float32)
    for j in range(DOT_CHUNKS):
        row = s * DOT_CHUNKS + j
        pltpu.sync_copy(uidx_hbm.at[row], idx_u_v)
        pltpu.sync_copy(midx_hbm.at[row], idx_m_v)
        du = pltpu.async_copy(uemb_hbm.at[idx_u_v], urows, sem_u)
        dm = pltpu.async_copy(memb_hbm.at[idx_m_v], mrows, sem_m)
        du.wait()
        dm.wait()

        def body(r, a):
            for q in range(D // L):
                a = a + urows[r, pl.ds(q * L, L)] * mrows[r, pl.ds(q * L, L)]
            return a

        acc = lax.fori_loop(0, CHUNK, body, acc)

    stage_v[...] = acc
    pltpu.sync_copy(stage_v, shared.at[s])
    plsc.subcore_barrier()
    pltpu.sync_copy(shared, part_v)
    tot = part_v[0]
    for q in range(1, NS):
        tot = tot + part_v[q]
    dot = tot[0]
    for q in range(1, L):
        dot = dot + tot[q]

    pltpu.sync_copy(gb_hbm, gb_v)
    base = dot + gb_v[...]
    for j in range(OUT_CHUNKS):
        row = ot * OUT_CHUNKS + j
        pltpu.sync_copy(uidx_hbm.at[row], idx_u_v)
        pltpu.sync_copy(midx_hbm.at[row], idx_m_v)
        du = pltpu.async_copy(ubias_hbm.at[idx_u_v], ub_v, sem_u)
        dm = pltpu.async_copy(mbias_hbm.at[idx_m_v], mb_v, sem_m)
        du.wait()
        dm.wait()
        for k in range(CHUNK // L):
            sl = pl.ds(k * L, L)
            out_v[pl.ds(j * CHUNK + k * L, L)] = ub_v[sl] + mb_v[sl] + base
    pltpu.sync_copy(out_v, out_hbm.at[pl.ds(ot * OUT_ROWS, OUT_ROWS)])


@jax.jit
def _run(uidx_r, midx_r, user_emb, movie_emb, ubias, mbias, gb_vec):
    mesh = plsc.VectorSubcoreMesh(core_axis_name="c", subcore_axis_name="s")
    f = functools.partial(
        pl.kernel,
        out_type=jax.ShapeDtypeStruct((B,), jnp.float32),
        mesh=mesh,
        compiler_params=pltpu.CompilerParams(use_tc_tiling_on_sc=False),
        scratch_types=[
            pltpu.VMEM((CHUNK,), jnp.int32),
            pltpu.VMEM((CHUNK,), jnp.int32),
            pltpu.VMEM((CHUNK, D), jnp.float32),
            pltpu.VMEM((CHUNK, D), jnp.float32),
            pltpu.VMEM((CHUNK,), jnp.float32),
            pltpu.VMEM((CHUNK,), jnp.float32),
            pltpu.VMEM((OUT_ROWS,), jnp.float32),
            pltpu.VMEM((L,), jnp.float32),
            pltpu.VMEM((L,), jnp.float32),
            pltpu.VMEM((NS, L), jnp.float32),
            pltpu.VMEM_SHARED((NS, L), jnp.float32),
            pltpu.SemaphoreType.DMA,
            pltpu.SemaphoreType.DMA,
        ],
    )(_sc_body)
    return f(uidx_r, midx_r, user_emb, movie_emb, ubias, mbias, gb_vec)


def kernel(inputs, user_emb, movie_emb, user_bias_table, movie_bias_table,
           global_bias):
    uidx = inputs[:, 0].reshape(B // CHUNK, CHUNK)
    midx = inputs[:, 1].reshape(B // CHUNK, CHUNK)
    ubias = user_bias_table.reshape(-1)
    mbias = movie_bias_table.reshape(-1)
    gb_vec = jnp.full((L,), global_bias, dtype=jnp.float32)
    return _run(uidx, midx, user_emb, movie_emb, ubias, mbias, gb_vec)


# speedup vs baseline: 1.4912x; 1.4912x over previous
import functools

import jax
import jax.numpy as jnp
from jax import lax
from jax.experimental import pallas as pl
from jax.experimental.pallas import tpu as pltpu, tpu_sc as plsc

NC = 2
NS = 16
L = 16
B = 16384
D = 64
ROWS_PER_TILE = B // (NC * NS)
GROUPS = ROWS_PER_TILE // L
NBUF = 2


def _dot_body(uidx_hbm, midx_hbm, ue_hbm, me_hbm, part_hbm,
              idx_u_v, idx_m_v, ubuf, mbuf, stage_v, sem_u, sem_m):
    c = lax.axis_index("c")
    s = lax.axis_index("s")
    ot = c * NS + s

    for j in range(ROWS_PER_TILE // 128):
        pltpu.sync_copy(uidx_hbm.at[ot * 4 + j], idx_u_v.at[pl.ds(j * 128, 128)])
        pltpu.sync_copy(midx_hbm.at[ot * 4 + j], idx_m_v.at[pl.ds(j * 128, 128)])

    def issue(g, b):
        iv_u = idx_u_v[pl.ds(g * L, L)]
        iv_m = idx_m_v[pl.ds(g * L, L)]
        for t in range(L):
            bu = pl.multiple_of((iv_u[t] >> 3) * 8, 8)
            bm = pl.multiple_of((iv_m[t] >> 3) * 8, 8)
            pltpu.async_copy(ue_hbm.at[pl.ds(bu, 8), :], ubuf.at[b, t], sem_u)
            pltpu.async_copy(me_hbm.at[pl.ds(bm, 8), :], mbuf.at[b, t], sem_m)

    def wait_group(b):
        for t in range(L):
            pltpu.make_async_copy(
                ue_hbm.at[pl.ds(0, 8), :], ubuf.at[b, t], sem_u).wait()
            pltpu.make_async_copy(
                me_hbm.at[pl.ds(0, 8), :], mbuf.at[b, t], sem_m).wait()

    for b in range(NBUF):
        issue(b, b)

    def body(i, acc):
        for b in range(NBUF):
            g = i * NBUF + b
            iv_u = idx_u_v[pl.ds(g * L, L)]
            iv_m = idx_m_v[pl.ds(g * L, L)]
            wait_group(b)
            for t in range(L):
                ru = iv_u[t] & 7
                rm = iv_m[t] & 7
                for q in range(D // L):
                    acc = acc + (ubuf[b, t, ru, pl.ds(q * L, L)]
                                 * mbuf[b, t, rm, pl.ds(q * L, L)])

            @pl.when(g + NBUF < GROUPS)
            def _():
                issue(g + NBUF, b)
        return acc

    acc = lax.fori_loop(0, GROUPS // NBUF, body, jnp.zeros((L,), jnp.float32))
    stage_v[...] = acc
    pltpu.sync_copy(stage_v, part_hbm.at[ot])


def _out_body(uidx_hbm, midx_hbm, ub_hbm, mb_hbm, gb_hbm, part_hbm, out_hbm,
              idx_u_v, idx_m_v, ubf, mbf, out_v, part_v, gb_v,
              sem_u, sem_m):
    c = lax.axis_index("c")
    s = lax.axis_index("s")
    ot = c * NS + s

    pltpu.sync_copy(part_hbm, part_v)
    pltpu.sync_copy(gb_hbm, gb_v)
    for j in range(ROWS_PER_TILE // 128):
        pltpu.sync_copy(uidx_hbm.at[ot * 4 + j], idx_u_v.at[pl.ds(j * 128, 128)])
        pltpu.sync_copy(midx_hbm.at[ot * 4 + j], idx_m_v.at[pl.ds(j * 128, 128)])
    descs = []
    for j in range(ROWS_PER_TILE // 128):
        sl = pl.ds(j * 128, 128)
        descs.append(pltpu.async_copy(
            ub_hbm.at[idx_u_v.at[sl]], ubf.at[sl], sem_u))
        descs.append(pltpu.async_copy(
            mb_hbm.at[idx_m_v.at[sl]], mbf.at[sl], sem_m))

    tot = part_v[0]
    for q in range(1, NC * NS):
        tot = tot + part_v[q]
    dot = tot[0]
    for q in range(1, L):
        dot = dot + tot[q]
    base = dot + gb_v[...]

    for d in descs:
        d.wait()
    for k in range(ROWS_PER_TILE // L):
        sl = pl.ds(k * L, L)
        out_v[sl] = ubf[sl] + mbf[sl] + base
    pltpu.sync_copy(out_v, out_hbm.at[pl.ds(ot * ROWS_PER_TILE, ROWS_PER_TILE)])


@jax.jit
def _run(uidx_r, midx_r, user_emb, movie_emb, ub2d, mb2d, gb_vec):
    mesh = plsc.VectorSubcoreMesh(core_axis_name="c", subcore_axis_name="s")
    part = functools.partial(
        pl.kernel,
        out_type=jax.ShapeDtypeStruct((NC * NS, L), jnp.float32),
        mesh=mesh,
        compiler_params=pltpu.CompilerParams(use_tc_tiling_on_sc=True),
        scratch_types=[
            pltpu.VMEM((ROWS_PER_TILE,), jnp.int32),
            pltpu.VMEM((ROWS_PER_TILE,), jnp.int32),
            pltpu.VMEM((NBUF, L, 8, D), jnp.float32),
            pltpu.VMEM((NBUF, L, 8, D), jnp.float32),
            pltpu.VMEM((L,), jnp.float32),
            pltpu.SemaphoreType.DMA,
            pltpu.SemaphoreType.DMA,
        ],
    )(_dot_body)(uidx_r, midx_r, user_emb, movie_emb)

    out = functools.partial(
        pl.kernel,
        out_type=jax.ShapeDtypeStruct((B,), jnp.float32),
        mesh=mesh,
        compiler_params=pltpu.CompilerParams(use_tc_tiling_on_sc=False),
        scratch_types=[
            pltpu.VMEM((ROWS_PER_TILE,), jnp.int32),
            pltpu.VMEM((ROWS_PER_TILE,), jnp.int32),
            pltpu.VMEM((ROWS_PER_TILE,), jnp.float32),
            pltpu.VMEM((ROWS_PER_TILE,), jnp.float32),
            pltpu.VMEM((ROWS_PER_TILE,), jnp.float32),
            pltpu.VMEM((NC * NS, L), jnp.float32),
            pltpu.VMEM((L,), jnp.float32),
            pltpu.SemaphoreType.DMA,
            pltpu.SemaphoreType.DMA,
        ],
    )(_out_body)(uidx_r, midx_r, ub2d, mb2d, gb_vec, part)
    return out


def kernel(inputs, user_emb, movie_emb, user_bias_table, movie_bias_table,
           global_bias):
    uidx = inputs[:, 0].reshape(B // 128, 128)
    midx = inputs[:, 1].reshape(B // 128, 128)
    gb_vec = jnp.full((L,), global_bias, dtype=jnp.float32)
    return _run(uidx, midx, user_emb, movie_emb,
                user_bias_table.reshape(-1), movie_bias_table.reshape(-1),
                gb_vec)
